# Initial kernel scaffold; baseline (speedup 1.0000x reference)
import jax
import jax.numpy as jnp
from jax.experimental import pallas as pl


def kernel(x, edge_index, W1, b1, W2, b2, W3, b3):
    raise NotImplementedError("write your pallas kernel here")


# speedup vs baseline: 32.7027x; 32.7027x over previous
import functools

import jax
import jax.numpy as jnp
from jax import lax
from jax.experimental import pallas as pl
from jax.experimental.pallas import tpu as pltpu
from jax.experimental.pallas import tpu_sc as plsc

N = 50000
E = 800000
F = 32
H = 64
NW = 32
NTILE = 16
CB = 128
KCH = 196
EPAD = NW * KCH * CB
NPAD = 50176
RPT = NPAD // NTILE
ZR = 196
NG = 14
NBLK = 8
RB = NPAD // NBLK

_MESH = plsc.VectorSubcoreMesh(core_axis_name="c", subcore_axis_name="s")
_SC_PARAMS = pltpu.CompilerParams(use_tc_tiling_on_sc=False)


def _mm(a, b):
    return lax.dot_general(a, b, (((1,), (0,)), ((), ())),
                           precision=lax.Precision.HIGHEST,
                           preferred_element_type=jnp.float32)


def _zero_1d(ref, n):
    @pl.loop(0, n // 16)
    def _(i):
        ref[pl.ds(i * 16, 16)] = jnp.zeros((16,), jnp.float32)


def _zero_2d(ref, n):
    @pl.loop(0, n)
    def _(r):
        z = jnp.zeros((16,), jnp.float32)
        ref[r, pl.ds(0, 16)] = z
        ref[r, pl.ds(16, 16)] = z


@functools.partial(
    pl.kernel,
    out_type=jax.ShapeDtypeStruct((2 * NPAD,), jnp.float32),
    mesh=_MESH,
    compiler_params=_SC_PARAMS,
    scratch_types=[
        pltpu.VMEM((KCH, CB), jnp.int32),
        pltpu.VMEM((CB,), jnp.float32),
        pltpu.VMEM((RPT,), jnp.float32),
        pltpu.VMEM_SHARED((NPAD,), jnp.float32),
    ],
)
def _sc_degree(dst_hbm, out_hbm, idst, ones, zb, dacc):
    cid = lax.axis_index("c")
    sid = lax.axis_index("s")
    wid = cid * NTILE + sid

    @pl.loop(0, CB // 16)
    def _(i):
        ones[pl.ds(i * 16, 16)] = jnp.ones((16,), jnp.float32)

    _zero_1d(zb, RPT)
    pltpu.sync_copy(zb, dacc.at[pl.ds(sid * RPT, RPT)])
    pltpu.sync_copy(dst_hbm.at[wid], idst)
    plsc.subcore_barrier()

    @pl.loop(0, KCH)
    def _(j):
        pltpu.sync_copy(ones, dacc.at[idst.at[j]], add=True)

    plsc.subcore_barrier()
    pltpu.sync_copy(dacc.at[pl.ds(sid * RPT, RPT)], zb)
    pltpu.sync_copy(zb, out_hbm.at[pl.ds(cid * NPAD + sid * RPT, RPT)])


@functools.partial(
    pl.kernel,
    out_type=(jax.ShapeDtypeStruct((2, NPAD, F), jnp.float32),
              jax.ShapeDtypeStruct((2 * NPAD,), jnp.float32)),
    mesh=_MESH,
    compiler_params=_SC_PARAMS,
    scratch_types=[
        pltpu.VMEM((NG, CB), jnp.int32),
        pltpu.VMEM((NG, CB), jnp.int32),
        pltpu.VMEM((CB, F), jnp.float32),
        pltpu.VMEM((CB,), jnp.float32),
        pltpu.VMEM((ZR, F), jnp.float32),
        pltpu.VMEM((RPT,), jnp.float32),
        pltpu.SemaphoreType.DMA,
        pltpu.SemaphoreType.DMA,
        pltpu.VMEM_SHARED((NPAD, F), jnp.float32),
        pltpu.VMEM_SHARED((NPAD,), jnp.float32),
    ],
)
def _sc_prop_c(g_hbm, dinv_hbm, src_hbm, dst_hbm, out_hbm, cout_hbm,
               isrc, idst, rows, cv, zb, zc, sem1, sem2, acc, cacc):
    cid = lax.axis_index("c")
    sid = lax.axis_index("s")
    wid = cid * NTILE + sid

    _zero_2d(zb, ZR)
    _zero_1d(zc, RPT)

    @pl.loop(0, RPT // ZR)
    def _(k):
        pltpu.sync_copy(zb, acc.at[pl.ds(sid * RPT + k * ZR, ZR)])

    pltpu.sync_copy(zc, cacc.at[pl.ds(sid * RPT, RPT)])
    plsc.subcore_barrier()

    @pl.loop(0, KCH // NG)
    def _(g):
        pltpu.sync_copy(src_hbm.at[wid, pl.ds(g * NG, NG)], isrc)
        pltpu.sync_copy(dst_hbm.at[wid, pl.ds(g * NG, NG)], idst)

        @pl.loop(0, NG)
        def _(j):
            d1 = pltpu.async_copy(g_hbm.at[isrc.at[j]], rows, sem1)
            d2 = pltpu.async_copy(dinv_hbm.at[idst.at[j]], cv, sem2)
            d1.wait()
            pltpu.sync_copy(rows, acc.at[idst.at[j]], add=True)
            d2.wait()
            pltpu.sync_copy(cv, cacc.at[isrc.at[j]], add=True)

    plsc.subcore_barrier()

    @pl.loop(0, RPT // ZR)
    def _(k):
        pltpu.sync_copy(acc.at[pl.ds(sid * RPT + k * ZR, ZR)], zb)
        pltpu.sync_copy(zb, out_hbm.at[cid, pl.ds(sid * RPT + k * ZR, ZR)])

    pltpu.sync_copy(cacc.at[pl.ds(sid * RPT, RPT)], zc)
    pltpu.sync_copy(zc, cout_hbm.at[pl.ds(cid * NPAD + sid * RPT, RPT)])


@functools.partial(
    pl.kernel,
    out_type=jax.ShapeDtypeStruct((2, NPAD, F), jnp.float32),
    mesh=_MESH,
    compiler_params=_SC_PARAMS,
    scratch_types=[
        pltpu.VMEM((NG, CB), jnp.int32),
        pltpu.VMEM((NG, CB), jnp.int32),
        pltpu.VMEM((CB, F), jnp.float32),
        pltpu.VMEM((ZR, F), jnp.float32),
        pltpu.SemaphoreType.DMA,
        pltpu.VMEM_SHARED((NPAD, F), jnp.float32),
    ],
)
def _sc_prop(g_hbm, src_hbm, dst_hbm, out_hbm, isrc, idst, rows, zb, sem1, acc):
    cid = lax.axis_index("c")
    sid = lax.axis_index("s")
    wid = cid * NTILE + sid

    _zero_2d(zb, ZR)

    @pl.loop(0, RPT // ZR)
    def _(k):
        pltpu.sync_copy(zb, acc.at[pl.ds(sid * RPT + k * ZR, ZR)])

    plsc.subcore_barrier()

    @pl.loop(0, KCH // NG)
    def _(g):
        pltpu.sync_copy(src_hbm.at[wid, pl.ds(g * NG, NG)], isrc)
        pltpu.sync_copy(dst_hbm.at[wid, pl.ds(g * NG, NG)], idst)

        @pl.loop(0, NG)
        def _(j):
            pltpu.async_copy(g_hbm.at[isrc.at[j]], rows, sem1).wait()
            pltpu.sync_copy(rows, acc.at[idst.at[j]], add=True)

    plsc.subcore_barrier()

    @pl.loop(0, RPT // ZR)
    def _(k):
        pltpu.sync_copy(acc.at[pl.ds(sid * RPT + k * ZR, ZR)], zb)
        pltpu.sync_copy(zb, out_hbm.at[cid, pl.ds(sid * RPT + k * ZR, ZR)])


def _tc1_body(dp_ref, x_ref, dinv_ref, g_ref):
    i = pl.program_id(0)
    dp = dp_ref[...]
    deg = dp[0] + dp[1] + 1.0
    row = i * RB + lax.broadcasted_iota(jnp.int32, (RB, 1), 0)
    dv = jnp.where(row < N, lax.rsqrt(deg), 0.0)
    dinv_ref[...] = dv
    g_ref[...] = x_ref[...] * dv


def _tc1(degp, x_p):
    return pl.pallas_call(
        _tc1_body,
        grid=(NBLK,),
        in_specs=[pl.BlockSpec((2, RB, 1), lambda i: (0, i, 0)),
                  pl.BlockSpec((RB, F), lambda i: (i, 0))],
        out_specs=[pl.BlockSpec((RB, 1), lambda i: (i, 0)),
                   pl.BlockSpec((RB, F), lambda i: (i, 0))],
        out_shape=[jax.ShapeDtypeStruct((NPAD, 1), jnp.float32),
                   jax.ShapeDtypeStruct((NPAD, F), jnp.float32)],
    )(degp, x_p)


def _tc2_body(ap_ref, g0_ref, dinv_ref, w1_ref, b1_ref, w2_ref, g2_ref):
    ap = ap_ref[...]
    dv = dinv_ref[...]
    p1 = dv * (ap[0] + ap[1] + g0_ref[...])
    h1 = jnp.maximum(_mm(p1, w1_ref[...]) + b1_ref[...], 0.0)
    g2_ref[...] = dv * _mm(h1, w2_ref[...])


def _tc2(accp, g0, dinv, W1, b1r, W2):
    return pl.pallas_call(
        _tc2_body,
        grid=(NBLK,),
        in_specs=[pl.BlockSpec((2, RB, F), lambda i: (0, i, 0)),
                  pl.BlockSpec((RB, F), lambda i: (i, 0)),
                  pl.BlockSpec((RB, 1), lambda i: (i, 0)),
                  pl.BlockSpec((F, H), lambda i: (0, 0)),
                  pl.BlockSpec((1, H), lambda i: (0, 0)),
                  pl.BlockSpec((H, F), lambda i: (0, 0))],
        out_specs=pl.BlockSpec((RB, F), lambda i: (i, 0)),
        out_shape=jax.ShapeDtypeStruct((NPAD, F), jnp.float32),
    )(accp, g0, dinv, W1, b1r, W2)


def _tc3_body(ap_ref, g2_ref, dinv_ref, cp_ref, b2_ref, w3_ref, b3_ref,
              o_ref, t_ref):
    i = pl.program_id(0)

    @pl.when(i == 0)
    def _():
        t_ref[...] = jnp.zeros((1, F), jnp.float32)

    ap = ap_ref[...]
    dv = dinv_ref[...]
    h2 = jnp.maximum(dv * (ap[0] + ap[1] + g2_ref[...]) + b2_ref[...], 0.0)
    cp = cp_ref[...]
    c = dv * (cp[0] + cp[1] + dv)
    t_ref[...] += jnp.sum(c * h2, axis=0, keepdims=True)

    @pl.when(i == NBLK - 1)
    def _():
        pooled = (jnp.sum(t_ref[...] * w3_ref[...], axis=1, keepdims=True)
                  + N * b3_ref[...])
        o_ref[...] = jax.nn.sigmoid(pooled)


def _tc3(accp, g2, dinv, cpart, b2r, w3r, b3r):
    return pl.pallas_call(
        _tc3_body,
        grid=(NBLK,),
        in_specs=[pl.BlockSpec((2, RB, F), lambda i: (0, i, 0)),
                  pl.BlockSpec((RB, F), lambda i: (i, 0)),
                  pl.BlockSpec((RB, 1), lambda i: (i, 0)),
                  pl.BlockSpec((2, RB, 1), lambda i: (0, i, 0)),
                  pl.BlockSpec((1, F), lambda i: (0, 0)),
                  pl.BlockSpec((1, F), lambda i: (0, 0)),
                  pl.BlockSpec((1, 1), lambda i: (0, 0))],
        out_specs=pl.BlockSpec((1, 1), lambda i: (0, 0)),
        out_shape=jax.ShapeDtypeStruct((1, 1), jnp.float32),
        scratch_shapes=[pltpu.VMEM((1, F), jnp.float32)],
    )(accp, g2, dinv, cpart, b2r, w3r, b3r)


def kernel(x, edge_index, W1, b1, W2, b2, W3, b3):
    src = edge_index[0]
    dst = edge_index[1]
    pad_s = jnp.full((EPAD - E,), NPAD - 1, jnp.int32)
    pad_d = jnp.full((EPAD - E,), NPAD - 2, jnp.int32)
    src_r = jnp.concatenate([src, pad_s]).reshape(NW, KCH, CB)
    dst_r = jnp.concatenate([dst, pad_d]).reshape(NW, KCH, CB)
    x_p = jnp.pad(x, ((0, NPAD - N), (0, 0)))

    degp = _sc_degree(dst_r)
    dinv, g0 = _tc1(degp.reshape(2, NPAD, 1), x_p)
    dinv_flat = dinv.reshape(NPAD)
    acc1, cpart = _sc_prop_c(g0, dinv_flat, src_r, dst_r)
    g2 = _tc2(acc1, g0, dinv, W1, b1.reshape(1, H), W2)
    acc2 = _sc_prop(g2, src_r, dst_r)
    out = _tc3(acc2, g2, dinv, cpart.reshape(2, NPAD, 1),
               b2.reshape(1, F), W3.reshape(1, F), b3.reshape(1, 1))
    return out.reshape(1)


# speedup vs baseline: 41.4280x; 1.2668x over previous
import functools

import jax
import jax.numpy as jnp
from jax import lax
from jax.experimental import pallas as pl
from jax.experimental.pallas import tpu as pltpu
from jax.experimental.pallas import tpu_sc as plsc

N = 50000
E = 800000
F = 32
H = 64
NW = 32
NTILE = 16
CB = 128
KCH = 196
EPAD = NW * KCH * CB
NPAD = 50176
RPT = NPAD // NTILE
ZR = 49
NG = 28
D = 4
NBLK = 8
RB = NPAD // NBLK

_MESH = plsc.VectorSubcoreMesh(core_axis_name="c", subcore_axis_name="s")
_SC_PARAMS = pltpu.CompilerParams(use_tc_tiling_on_sc=False)


def _mm(a, b):
    return lax.dot_general(a, b, (((1,), (0,)), ((), ())),
                           precision=lax.Precision.HIGHEST,
                           preferred_element_type=jnp.float32)


def _zero_1d(ref, n):
    @pl.loop(0, n // 16)
    def _(i):
        ref[pl.ds(i * 16, 16)] = jnp.zeros((16,), jnp.float32)


def _zero_2d(ref, n):
    @pl.loop(0, n)
    def _(r):
        z = jnp.zeros((16,), jnp.float32)
        ref[r, pl.ds(0, 16)] = z
        ref[r, pl.ds(16, 16)] = z


@functools.partial(
    pl.kernel,
    out_type=jax.ShapeDtypeStruct((2 * NPAD,), jnp.float32),
    mesh=_MESH,
    compiler_params=_SC_PARAMS,
    scratch_types=[
        pltpu.VMEM((NG, CB), jnp.int32),
        pltpu.VMEM((CB,), jnp.float32),
        pltpu.VMEM((RPT,), jnp.float32),
        pltpu.SemaphoreType.DMA,
        pltpu.SemaphoreType.DMA,
        pltpu.SemaphoreType.DMA,
        pltpu.SemaphoreType.DMA,
        pltpu.VMEM_SHARED((NPAD,), jnp.float32),
    ],
)
def _sc_degree(dst_hbm, out_hbm, idst, ones, zb, s0, s1, s2, s3, dacc):
    cid = lax.axis_index("c")
    sid = lax.axis_index("s")
    wid = cid * NTILE + sid

    @pl.loop(0, CB // 16)
    def _(i):
        ones[pl.ds(i * 16, 16)] = jnp.ones((16,), jnp.float32)

    _zero_1d(zb, RPT)
    pltpu.sync_copy(zb, dacc.at[pl.ds(sid * RPT, RPT)])
    plsc.subcore_barrier()

    ssem = (s0, s1, s2, s3)

    @pl.loop(0, KCH // NG)
    def _(gi):
        pltpu.sync_copy(dst_hbm.at[wid, pl.ds(gi * NG, NG)], idst)

        @pl.loop(0, NG // D)
        def _(bi):
            ds = [pltpu.async_copy(ones, dacc.at[idst.at[bi * D + k]],
                                   ssem[k], add=True) for k in range(D)]
            for d in ds:
                d.wait()

    plsc.subcore_barrier()
    pltpu.sync_copy(dacc.at[pl.ds(sid * RPT, RPT)], zb)
    pltpu.sync_copy(zb, out_hbm.at[pl.ds(cid * NPAD + sid * RPT, RPT)])


@functools.partial(
    pl.kernel,
    out_type=(jax.ShapeDtypeStruct((2, NPAD, F), jnp.float32),
              jax.ShapeDtypeStruct((2 * NPAD,), jnp.float32)),
    mesh=_MESH,
    compiler_params=_SC_PARAMS,
    scratch_types=[
        pltpu.VMEM((NG, CB), jnp.int32),
        pltpu.VMEM((NG, CB), jnp.int32),
        pltpu.VMEM((D, CB, F), jnp.float32),
        pltpu.VMEM((D, CB), jnp.float32),
        pltpu.VMEM((ZR, F), jnp.float32),
        pltpu.VMEM((RPT // 4,), jnp.float32),
        [pltpu.SemaphoreType.DMA] * D,
        [pltpu.SemaphoreType.DMA] * D,
        [pltpu.SemaphoreType.DMA] * D,
        [pltpu.SemaphoreType.DMA] * D,
        pltpu.VMEM_SHARED((NPAD, F), jnp.float32),
        pltpu.VMEM_SHARED((NPAD,), jnp.float32),
    ],
)
def _sc_prop_c(g_hbm, dinv_hbm, src_hbm, dst_hbm, out_hbm, cout_hbm,
               isrc, idst, rows, cv, zb, zc, gsem, ssem, cgsem, cssem,
               acc, cacc):
    cid = lax.axis_index("c")
    sid = lax.axis_index("s")
    wid = cid * NTILE + sid

    _zero_2d(zb, ZR)
    _zero_1d(zc, RPT // 4)

    @pl.loop(0, RPT // ZR)
    def _(k):
        pltpu.sync_copy(zb, acc.at[pl.ds(sid * RPT + k * ZR, ZR)])

    @pl.loop(0, 4)
    def _(q):
        pltpu.sync_copy(zc, cacc.at[pl.ds(sid * RPT + q * (RPT // 4), RPT // 4)])
    plsc.subcore_barrier()

    @pl.loop(0, KCH // NG)
    def _(gi):
        pltpu.sync_copy(src_hbm.at[wid, pl.ds(gi * NG, NG)], isrc)
        pltpu.sync_copy(dst_hbm.at[wid, pl.ds(gi * NG, NG)], idst)

        @pl.loop(0, NG // D)
        def _(bi):
            gd = [(pltpu.async_copy(g_hbm.at[isrc.at[bi * D + k]],
                                    rows.at[k], gsem[k]),
                   pltpu.async_copy(dinv_hbm.at[idst.at[bi * D + k]],
                                    cv.at[k], cgsem[k]))
                  for k in range(D)]
            sd = []
            for k in range(D):
                gd[k][0].wait()
                sd.append(pltpu.async_copy(
                    rows.at[k], acc.at[idst.at[bi * D + k]], ssem[k],
                    add=True))
                gd[k][1].wait()
                sd.append(pltpu.async_copy(
                    cv.at[k], cacc.at[isrc.at[bi * D + k]], cssem[k],
                    add=True))
            for d in sd:
                d.wait()

    plsc.subcore_barrier()

    @pl.loop(0, RPT // ZR)
    def _(k):
        pltpu.sync_copy(acc.at[pl.ds(sid * RPT + k * ZR, ZR)], zb)
        pltpu.sync_copy(zb, out_hbm.at[cid, pl.ds(sid * RPT + k * ZR, ZR)])

    @pl.loop(0, 4)
    def _(q):
        pltpu.sync_copy(cacc.at[pl.ds(sid * RPT + q * (RPT // 4), RPT // 4)], zc)
        pltpu.sync_copy(
            zc, cout_hbm.at[pl.ds(cid * NPAD + sid * RPT + q * (RPT // 4),
                                  RPT // 4)])


@functools.partial(
    pl.kernel,
    out_type=jax.ShapeDtypeStruct((2, NPAD, F), jnp.float32),
    mesh=_MESH,
    compiler_params=_SC_PARAMS,
    scratch_types=[
        pltpu.VMEM((NG, CB), jnp.int32),
        pltpu.VMEM((NG, CB), jnp.int32),
        pltpu.VMEM((D, CB, F), jnp.float32),
        pltpu.VMEM((ZR, F), jnp.float32),
        [pltpu.SemaphoreType.DMA] * D,
        [pltpu.SemaphoreType.DMA] * D,
        pltpu.VMEM_SHARED((NPAD, F), jnp.float32),
    ],
)
def _sc_prop(g_hbm, src_hbm, dst_hbm, out_hbm, isrc, idst, rows, zb,
             gsem, ssem, acc):
    cid = lax.axis_index("c")
    sid = lax.axis_index("s")
    wid = cid * NTILE + sid

    _zero_2d(zb, ZR)

    @pl.loop(0, RPT // ZR)
    def _(k):
        pltpu.sync_copy(zb, acc.at[pl.ds(sid * RPT + k * ZR, ZR)])

    plsc.subcore_barrier()

    @pl.loop(0, KCH // NG)
    def _(gi):
        pltpu.sync_copy(src_hbm.at[wid, pl.ds(gi * NG, NG)], isrc)
        pltpu.sync_copy(dst_hbm.at[wid, pl.ds(gi * NG, NG)], idst)

        @pl.loop(0, NG // D)
        def _(bi):
            gd = [pltpu.async_copy(g_hbm.at[isrc.at[bi * D + k]],
                                   rows.at[k], gsem[k])
                  for k in range(D)]
            sd = []
            for k in range(D):
                gd[k].wait()
                sd.append(pltpu.async_copy(
                    rows.at[k], acc.at[idst.at[bi * D + k]], ssem[k],
                    add=True))
            for d in sd:
                d.wait()

    plsc.subcore_barrier()

    @pl.loop(0, RPT // ZR)
    def _(k):
        pltpu.sync_copy(acc.at[pl.ds(sid * RPT + k * ZR, ZR)], zb)
        pltpu.sync_copy(zb, out_hbm.at[cid, pl.ds(sid * RPT + k * ZR, ZR)])


def _tc1_body(dp_ref, x_ref, dinv_ref, g_ref):
    i = pl.program_id(0)
    dp = dp_ref[...]
    deg = dp[0] + dp[1] + 1.0
    row = i * RB + lax.broadcasted_iota(jnp.int32, (RB, 1), 0)
    dv = jnp.where(row < N, lax.rsqrt(deg), 0.0)
    dinv_ref[...] = dv
    g_ref[...] = x_ref[...] * dv


def _tc1(degp, x_p):
    return pl.pallas_call(
        _tc1_body,
        grid=(NBLK,),
        in_specs=[pl.BlockSpec((2, RB, 1), lambda i: (0, i, 0)),
                  pl.BlockSpec((RB, F), lambda i: (i, 0))],
        out_specs=[pl.BlockSpec((RB, 1), lambda i: (i, 0)),
                   pl.BlockSpec((RB, F), lambda i: (i, 0))],
        out_shape=[jax.ShapeDtypeStruct((NPAD, 1), jnp.float32),
                   jax.ShapeDtypeStruct((NPAD, F), jnp.float32)],
    )(degp, x_p)


def _tc2_body(ap_ref, g0_ref, dinv_ref, w1_ref, b1_ref, w2_ref, g2_ref):
    ap = ap_ref[...]
    dv = dinv_ref[...]
    p1 = dv * (ap[0] + ap[1] + g0_ref[...])
    h1 = jnp.maximum(_mm(p1, w1_ref[...]) + b1_ref[...], 0.0)
    g2_ref[...] = dv * _mm(h1, w2_ref[...])


def _tc2(accp, g0, dinv, W1, b1r, W2):
    return pl.pallas_call(
        _tc2_body,
        grid=(NBLK,),
        in_specs=[pl.BlockSpec((2, RB, F), lambda i: (0, i, 0)),
                  pl.BlockSpec((RB, F), lambda i: (i, 0)),
                  pl.BlockSpec((RB, 1), lambda i: (i, 0)),
                  pl.BlockSpec((F, H), lambda i: (0, 0)),
                  pl.BlockSpec((1, H), lambda i: (0, 0)),
                  pl.BlockSpec((H, F), lambda i: (0, 0))],
        out_specs=pl.BlockSpec((RB, F), lambda i: (i, 0)),
        out_shape=jax.ShapeDtypeStruct((NPAD, F), jnp.float32),
    )(accp, g0, dinv, W1, b1r, W2)


def _tc3_body(ap_ref, g2_ref, dinv_ref, cp_ref, b2_ref, w3_ref, b3_ref,
              o_ref, t_ref):
    i = pl.program_id(0)

    @pl.when(i == 0)
    def _():
        t_ref[...] = jnp.zeros((1, F), jnp.float32)

    ap = ap_ref[...]
    dv = dinv_ref[...]
    h2 = jnp.maximum(dv * (ap[0] + ap[1] + g2_ref[...]) + b2_ref[...], 0.0)
    cp = cp_ref[...]
    c = dv * (cp[0] + cp[1] + dv)
    t_ref[...] += jnp.sum(c * h2, axis=0, keepdims=True)

    @pl.when(i == NBLK - 1)
    def _():
        pooled = (jnp.sum(t_ref[...] * w3_ref[...], axis=1, keepdims=True)
                  + N * b3_ref[...])
        o_ref[...] = jax.nn.sigmoid(pooled)


def _tc3(accp, g2, dinv, cpart, b2r, w3r, b3r):
    return pl.pallas_call(
        _tc3_body,
        grid=(NBLK,),
        in_specs=[pl.BlockSpec((2, RB, F), lambda i: (0, i, 0)),
                  pl.BlockSpec((RB, F), lambda i: (i, 0)),
                  pl.BlockSpec((RB, 1), lambda i: (i, 0)),
                  pl.BlockSpec((2, RB, 1), lambda i: (0, i, 0)),
                  pl.BlockSpec((1, F), lambda i: (0, 0)),
                  pl.BlockSpec((1, F), lambda i: (0, 0)),
                  pl.BlockSpec((1, 1), lambda i: (0, 0))],
        out_specs=pl.BlockSpec((1, 1), lambda i: (0, 0)),
        out_shape=jax.ShapeDtypeStruct((1, 1), jnp.float32),
        scratch_shapes=[pltpu.VMEM((1, F), jnp.float32)],
    )(accp, g2, dinv, cpart, b2r, w3r, b3r)


def kernel(x, edge_index, W1, b1, W2, b2, W3, b3):
    src = edge_index[0]
    dst = edge_index[1]
    pad_s = jnp.full((EPAD - E,), NPAD - 1, jnp.int32)
    pad_d = jnp.full((EPAD - E,), NPAD - 2, jnp.int32)
    src_r = jnp.concatenate([src, pad_s]).reshape(NW, KCH, CB)
    dst_r = jnp.concatenate([dst, pad_d]).reshape(NW, KCH, CB)
    x_p = jnp.pad(x, ((0, NPAD - N), (0, 0)))

    degp = _sc_degree(dst_r)
    dinv, g0 = _tc1(degp.reshape(2, NPAD, 1), x_p)
    dinv_flat = dinv.reshape(NPAD)
    acc1, cpart = _sc_prop_c(g0, dinv_flat, src_r, dst_r)
    g2 = _tc2(acc1, g0, dinv, W1, b1.reshape(1, H), W2)
    acc2 = _sc_prop(g2, src_r, dst_r)
    out = _tc3(acc2, g2, dinv, cpart.reshape(2, NPAD, 1),
               b2.reshape(1, F), W3.reshape(1, F), b3.reshape(1, 1))
    return out.reshape(1)


# speedup vs baseline: 48.6576x; 1.1745x over previous
import functools

import jax
import jax.numpy as jnp
from jax import lax
from jax.experimental import pallas as pl
from jax.experimental.pallas import tpu as pltpu
from jax.experimental.pallas import tpu_sc as plsc

N = 50000
E = 800000
F = 32
H = 64
NW = 32
NTILE = 16
CB = 128
KCH = 196
EPAD = NW * KCH * CB
NPAD = 50176
RPT = NPAD // NTILE
ZR = 49
NG = 28
D = 4
NBLK = 7
RB = NPAD // NBLK

_MESH = plsc.VectorSubcoreMesh(core_axis_name="c", subcore_axis_name="s")
_SC_PARAMS = pltpu.CompilerParams(use_tc_tiling_on_sc=False)


def _mm(a, b):
    return lax.dot_general(a, b, (((1,), (0,)), ((), ())),
                           precision=lax.Precision.HIGHEST,
                           preferred_element_type=jnp.float32)


def _zero_1d(ref, n):
    @pl.loop(0, n // 16)
    def _(i):
        ref[pl.ds(i * 16, 16)] = jnp.zeros((16,), jnp.float32)


def _zero_2d(ref, n):
    @pl.loop(0, n)
    def _(r):
        z = jnp.zeros((16,), jnp.float32)
        ref[r, pl.ds(0, 16)] = z
        ref[r, pl.ds(16, 16)] = z


@functools.partial(
    pl.kernel,
    out_type=jax.ShapeDtypeStruct((2 * NPAD,), jnp.float32),
    mesh=_MESH,
    compiler_params=_SC_PARAMS,
    scratch_types=[
        pltpu.VMEM((NG, CB), jnp.int32),
        pltpu.VMEM((CB,), jnp.float32),
        pltpu.VMEM((RPT,), jnp.float32),
        pltpu.SemaphoreType.DMA,
        pltpu.SemaphoreType.DMA,
        pltpu.SemaphoreType.DMA,
        pltpu.SemaphoreType.DMA,
        pltpu.VMEM_SHARED((NPAD,), jnp.float32),
    ],
)
def _sc_degree(dst_hbm, out_hbm, idst, ones, zb, s0, s1, s2, s3, dacc):
    cid = lax.axis_index("c")
    sid = lax.axis_index("s")
    wid = cid * NTILE + sid

    @pl.loop(0, CB // 16)
    def _(i):
        ones[pl.ds(i * 16, 16)] = jnp.ones((16,), jnp.float32)

    _zero_1d(zb, RPT)
    pltpu.sync_copy(zb, dacc.at[pl.ds(sid * RPT, RPT)])
    plsc.subcore_barrier()

    ssem = (s0, s1, s2, s3)

    @pl.loop(0, KCH // NG)
    def _(gi):
        pltpu.sync_copy(dst_hbm.at[wid, pl.ds(gi * NG, NG)], idst)

        @pl.loop(0, NG // D)
        def _(bi):
            ds = [pltpu.async_copy(ones, dacc.at[idst.at[bi * D + k]],
                                   ssem[k], add=True) for k in range(D)]
            for d in ds:
                d.wait()

    plsc.subcore_barrier()
    pltpu.sync_copy(dacc.at[pl.ds(sid * RPT, RPT)], zb)
    pltpu.sync_copy(zb, out_hbm.at[pl.ds(cid * NPAD + sid * RPT, RPT)])


@functools.partial(
    pl.kernel,
    out_type=(jax.ShapeDtypeStruct((2, NPAD, F), jnp.float32),
              jax.ShapeDtypeStruct((2 * NPAD,), jnp.float32)),
    mesh=_MESH,
    compiler_params=_SC_PARAMS,
    scratch_types=[
        pltpu.VMEM((NG, CB), jnp.int32),
        pltpu.VMEM((NG, CB), jnp.int32),
        pltpu.VMEM((D, CB, F), jnp.float32),
        pltpu.VMEM((D, CB), jnp.float32),
        pltpu.VMEM((ZR, F), jnp.float32),
        pltpu.VMEM((RPT // 4,), jnp.float32),
        [pltpu.SemaphoreType.DMA] * D,
        [pltpu.SemaphoreType.DMA] * D,
        [pltpu.SemaphoreType.DMA] * D,
        [pltpu.SemaphoreType.DMA] * D,
        pltpu.VMEM_SHARED((NPAD, F), jnp.float32),
        pltpu.VMEM_SHARED((NPAD,), jnp.float32),
    ],
)
def _sc_prop_c(g_hbm, dinv_hbm, src_hbm, dst_hbm, out_hbm, cout_hbm,
               isrc, idst, rows, cv, zb, zc, gsem, ssem, cgsem, cssem,
               acc, cacc):
    cid = lax.axis_index("c")
    sid = lax.axis_index("s")
    wid = cid * NTILE + sid

    _zero_2d(zb, ZR)
    _zero_1d(zc, RPT // 4)

    @pl.loop(0, RPT // ZR)
    def _(k):
        pltpu.sync_copy(zb, acc.at[pl.ds(sid * RPT + k * ZR, ZR)])

    @pl.loop(0, 4)
    def _(q):
        pltpu.sync_copy(zc, cacc.at[pl.ds(sid * RPT + q * (RPT // 4), RPT // 4)])
    plsc.subcore_barrier()

    @pl.loop(0, KCH // NG)
    def _(gi):
        pltpu.sync_copy(src_hbm.at[wid, pl.ds(gi * NG, NG)], isrc)
        pltpu.sync_copy(dst_hbm.at[wid, pl.ds(gi * NG, NG)], idst)

        @pl.loop(0, NG // D)
        def _(bi):
            gd = [(pltpu.async_copy(g_hbm.at[isrc.at[bi * D + k]],
                                    rows.at[k], gsem[k]),
                   pltpu.async_copy(dinv_hbm.at[idst.at[bi * D + k]],
                                    cv.at[k], cgsem[k]))
                  for k in range(D)]
            sd = []
            for k in range(D):
                gd[k][0].wait()
                sd.append(pltpu.async_copy(
                    rows.at[k], acc.at[idst.at[bi * D + k]], ssem[k],
                    add=True))
                gd[k][1].wait()
                sd.append(pltpu.async_copy(
                    cv.at[k], cacc.at[isrc.at[bi * D + k]], cssem[k],
                    add=True))
            for d in sd:
                d.wait()

    plsc.subcore_barrier()

    @pl.loop(0, RPT // ZR)
    def _(k):
        pltpu.sync_copy(acc.at[pl.ds(sid * RPT + k * ZR, ZR)], zb)
        pltpu.sync_copy(zb, out_hbm.at[cid, pl.ds(sid * RPT + k * ZR, ZR)])

    @pl.loop(0, 4)
    def _(q):
        pltpu.sync_copy(cacc.at[pl.ds(sid * RPT + q * (RPT // 4), RPT // 4)], zc)
        pltpu.sync_copy(
            zc, cout_hbm.at[pl.ds(cid * NPAD + sid * RPT + q * (RPT // 4),
                                  RPT // 4)])


@functools.partial(
    pl.kernel,
    out_type=jax.ShapeDtypeStruct((2, NPAD, F), jnp.float32),
    mesh=_MESH,
    compiler_params=_SC_PARAMS,
    scratch_types=[
        pltpu.VMEM((NG, CB), jnp.int32),
        pltpu.VMEM((NG, CB), jnp.int32),
        pltpu.VMEM((D, CB, F), jnp.float32),
        pltpu.VMEM((ZR, F), jnp.float32),
        [pltpu.SemaphoreType.DMA] * D,
        [pltpu.SemaphoreType.DMA] * D,
        pltpu.VMEM_SHARED((NPAD, F), jnp.float32),
    ],
)
def _sc_prop(g_hbm, src_hbm, dst_hbm, out_hbm, isrc, idst, rows, zb,
             gsem, ssem, acc):
    cid = lax.axis_index("c")
    sid = lax.axis_index("s")
    wid = cid * NTILE + sid

    _zero_2d(zb, ZR)

    @pl.loop(0, RPT // ZR)
    def _(k):
        pltpu.sync_copy(zb, acc.at[pl.ds(sid * RPT + k * ZR, ZR)])

    plsc.subcore_barrier()

    @pl.loop(0, KCH // NG)
    def _(gi):
        pltpu.sync_copy(src_hbm.at[wid, pl.ds(gi * NG, NG)], isrc)
        pltpu.sync_copy(dst_hbm.at[wid, pl.ds(gi * NG, NG)], idst)

        @pl.loop(0, NG // D)
        def _(bi):
            gd = [pltpu.async_copy(g_hbm.at[isrc.at[bi * D + k]],
                                   rows.at[k], gsem[k])
                  for k in range(D)]
            sd = []
            for k in range(D):
                gd[k].wait()
                sd.append(pltpu.async_copy(
                    rows.at[k], acc.at[idst.at[bi * D + k]], ssem[k],
                    add=True))
            for d in sd:
                d.wait()

    plsc.subcore_barrier()

    @pl.loop(0, RPT // ZR)
    def _(k):
        pltpu.sync_copy(acc.at[pl.ds(sid * RPT + k * ZR, ZR)], zb)
        pltpu.sync_copy(zb, out_hbm.at[cid, pl.ds(sid * RPT + k * ZR, ZR)])


def _tc1_body(dp0_ref, dp1_ref, x_ref, dinv_ref, dinv32_ref, g_ref):
    i = pl.program_id(0)
    deg = dp0_ref[...] + dp1_ref[...] + 1.0
    row = i * RB + lax.broadcasted_iota(jnp.int32, (RB,), 0)
    dv = jnp.where(row < N, lax.rsqrt(deg), 0.0)
    dinv_ref[...] = dv
    dv32 = jnp.broadcast_to(dv.reshape(RB, 1), (RB, F))
    dinv32_ref[...] = dv32
    g_ref[...] = jnp.where(dv32 > 0.0, x_ref[...] * dv32, 0.0)


def _tc1(degp, x):
    return pl.pallas_call(
        _tc1_body,
        grid=(NBLK,),
        in_specs=[pl.BlockSpec((RB,), lambda i: (i,)),
                  pl.BlockSpec((RB,), lambda i: (i + NBLK,)),
                  pl.BlockSpec((RB, F), lambda i: (i, 0))],
        out_specs=[pl.BlockSpec((RB,), lambda i: (i,)),
                   pl.BlockSpec((RB, F), lambda i: (i, 0)),
                   pl.BlockSpec((RB, F), lambda i: (i, 0))],
        out_shape=[jax.ShapeDtypeStruct((NPAD,), jnp.float32),
                   jax.ShapeDtypeStruct((NPAD, F), jnp.float32),
                   jax.ShapeDtypeStruct((NPAD, F), jnp.float32)],
    )(degp, degp, x)


def _tc2_body(ap_ref, g0_ref, dv32_ref, w1_ref, b1_ref, w2_ref, g2_ref):
    ap = ap_ref[...]
    dv32 = dv32_ref[...]
    p1 = dv32 * (ap[0] + ap[1] + g0_ref[...])
    h1 = jnp.maximum(_mm(p1, w1_ref[...]) + b1_ref[...], 0.0)
    g2_ref[...] = dv32 * _mm(h1, w2_ref[...])


def _tc2(accp, g0, dinv32, W1, b1r, W2):
    return pl.pallas_call(
        _tc2_body,
        grid=(NBLK,),
        in_specs=[pl.BlockSpec((2, RB, F), lambda i: (0, i, 0)),
                  pl.BlockSpec((RB, F), lambda i: (i, 0)),
                  pl.BlockSpec((RB, F), lambda i: (i, 0)),
                  pl.BlockSpec((F, H), lambda i: (0, 0)),
                  pl.BlockSpec((1, H), lambda i: (0, 0)),
                  pl.BlockSpec((H, F), lambda i: (0, 0))],
        out_specs=pl.BlockSpec((RB, F), lambda i: (i, 0)),
        out_shape=jax.ShapeDtypeStruct((NPAD, F), jnp.float32),
    )(accp, g0, dinv32, W1, b1r, W2)


def _tc3_body(ap_ref, g2_ref, dv32_ref, dv_ref, cp0_ref, cp1_ref, b2_ref,
              w3_ref, b3_ref, o_ref, t_ref):
    i = pl.program_id(0)

    @pl.when(i == 0)
    def _():
        t_ref[...] = jnp.zeros((1, F), jnp.float32)

    ap = ap_ref[...]
    dv32 = dv32_ref[...]
    h2 = jnp.maximum(dv32 * (ap[0] + ap[1] + g2_ref[...]) + b2_ref[...], 0.0)
    dv = dv_ref[...]
    c = dv * (cp0_ref[...] + cp1_ref[...] + dv)
    t_ref[...] += jnp.sum(c.reshape(RB, 1) * h2, axis=0, keepdims=True)

    @pl.when(i == NBLK - 1)
    def _():
        pooled = (jnp.sum(t_ref[...] * w3_ref[...], axis=1, keepdims=True)
                  + N * b3_ref[...])
        o_ref[...] = jax.nn.sigmoid(pooled)


def _tc3(accp, g2, dinv32, dinv, cpart, b2r, w3r, b3r):
    return pl.pallas_call(
        _tc3_body,
        grid=(NBLK,),
        in_specs=[pl.BlockSpec((2, RB, F), lambda i: (0, i, 0)),
                  pl.BlockSpec((RB, F), lambda i: (i, 0)),
                  pl.BlockSpec((RB, F), lambda i: (i, 0)),
                  pl.BlockSpec((RB,), lambda i: (i,)),
                  pl.BlockSpec((RB,), lambda i: (i,)),
                  pl.BlockSpec((RB,), lambda i: (i + NBLK,)),
                  pl.BlockSpec((1, F), lambda i: (0, 0)),
                  pl.BlockSpec((1, F), lambda i: (0, 0)),
                  pl.BlockSpec((1, 1), lambda i: (0, 0))],
        out_specs=pl.BlockSpec((1, 1), lambda i: (0, 0)),
        out_shape=jax.ShapeDtypeStruct((1, 1), jnp.float32),
        scratch_shapes=[pltpu.VMEM((1, F), jnp.float32)],
    )(accp, g2, dinv32, dinv, cpart, cpart, b2r, w3r, b3r)


def kernel(x, edge_index, W1, b1, W2, b2, W3, b3):
    src = edge_index[0]
    dst = edge_index[1]
    pad_s = jnp.full((EPAD - E,), NPAD - 1, jnp.int32)
    pad_d = jnp.full((EPAD - E,), NPAD - 2, jnp.int32)
    src_r = jnp.concatenate([src, pad_s]).reshape(NW, KCH, CB)
    dst_r = jnp.concatenate([dst, pad_d]).reshape(NW, KCH, CB)

    degp = _sc_degree(dst_r)
    dinv, dinv32, g0 = _tc1(degp, x)
    acc1, cpart = _sc_prop_c(g0, dinv, src_r, dst_r)
    g2 = _tc2(acc1, g0, dinv32, W1, b1.reshape(1, H), W2)
    acc2 = _sc_prop(g2, src_r, dst_r)
    out = _tc3(acc2, g2, dinv32, dinv, cpart,
               b2.reshape(1, F), W3.reshape(1, F), b3.reshape(1, 1))
    return out.reshape(1)
